# baseline (speedup 1.0000x reference)
import functools

import jax
import jax.numpy as jnp
from jax import lax
from jax.experimental import pallas as pl
from jax.experimental.pallas import tpu as pltpu
from jax.experimental.pallas import tpu_sc as plsc

_N = 10000
_D = 128
_H = 128
_OUT = 192
_E = 320000

_NC = 2
_NS = 16
_NW = _NC * _NS
_CHUNK = 128
_CHUNKS = 80
_EPAD = _NW * _CHUNKS * _CHUNK
_DUMP = _N
_NACC = _N + 8
_HCH = 16
_STAGES = _CHUNKS // _HCH
_ZROWS = 400
_ZTASKS = _N // _ZROWS

def _seg_sum_body(x_hbm, src_hbm, dst_hbm, zblk_hbm, out_hbm,
                  acc, src_v, dst_v, rows_a, rows_b,
                  sem_a, sem_b, sem_sa, sem_sb):
    cid = lax.axis_index("c")
    sid = lax.axis_index("s")
    wid = cid * _NS + sid

    pltpu.sync_copy(zblk_hbm, acc.at[pl.ds(sid * _ZROWS, _ZROWS)])

    @pl.when(sid < _ZTASKS - _NS)
    def _():
        pltpu.sync_copy(zblk_hbm,
                        acc.at[pl.ds((sid + _NS) * _ZROWS, _ZROWS)])

    plsc.subcore_barrier()

    for h in range(_STAGES):
        pltpu.sync_copy(src_hbm.at[wid, pl.ds(h * _HCH, _HCH)], src_v)
        pltpu.sync_copy(dst_hbm.at[wid, pl.ds(h * _HCH, _HCH)], dst_v)
        pltpu.async_copy(x_hbm.at[src_v.at[0]], rows_a, sem_a)
        pltpu.async_copy(x_hbm.at[src_v.at[1]], rows_b, sem_b)

        def body(i, carry):
            j = i * 2
            pltpu.make_async_copy(x_hbm.at[src_v.at[j]], rows_a, sem_a).wait()
            pltpu.async_copy(rows_a, acc.at[dst_v.at[j]], sem_sa, add=True)

            @pl.when(j + 2 < _HCH)
            def _():
                pltpu.make_async_copy(rows_a, acc.at[dst_v.at[j]],
                                      sem_sa).wait()
                pltpu.async_copy(x_hbm.at[src_v.at[j + 2]], rows_a, sem_a)

            pltpu.make_async_copy(x_hbm.at[src_v.at[j + 1]], rows_b,
                                  sem_b).wait()
            pltpu.async_copy(rows_b, acc.at[dst_v.at[j + 1]], sem_sb,
                             add=True)

            @pl.when(j + 3 < _HCH)
            def _():
                pltpu.make_async_copy(rows_b, acc.at[dst_v.at[j + 1]],
                                      sem_sb).wait()
                pltpu.async_copy(x_hbm.at[src_v.at[j + 3]], rows_b, sem_b)

            return carry

        lax.fori_loop(0, _HCH // 2, body, 0)
        pltpu.make_async_copy(rows_a, acc.at[dst_v.at[0]], sem_sa).wait()
        pltpu.make_async_copy(rows_b, acc.at[dst_v.at[0]], sem_sb).wait()

    plsc.subcore_barrier()

    pltpu.sync_copy(acc.at[pl.ds(sid * _ZROWS, _ZROWS)],
                    out_hbm.at[cid, pl.ds(sid * _ZROWS, _ZROWS)])

    @pl.when(sid < _ZTASKS - _NS)
    def _():
        pltpu.sync_copy(acc.at[pl.ds((sid + _NS) * _ZROWS, _ZROWS)],
                        out_hbm.at[cid, pl.ds((sid + _NS) * _ZROWS, _ZROWS)])


@functools.cache
def _seg_sum():
    mesh = plsc.VectorSubcoreMesh(core_axis_name="c", subcore_axis_name="s",
                                  num_cores=_NC, num_subcores=_NS)
    return pl.kernel(
        _seg_sum_body,
        out_type=jax.ShapeDtypeStruct((_NC, _N, _D), jnp.float32),
        mesh=mesh,
        scratch_types=[
            pltpu.VMEM_SHARED((_NACC, _D), jnp.float32),
            pltpu.VMEM((_HCH, _CHUNK), jnp.int32),
            pltpu.VMEM((_HCH, _CHUNK), jnp.int32),
            pltpu.VMEM((_CHUNK, _D), jnp.float32),
            pltpu.VMEM((_CHUNK, _D), jnp.float32),
            pltpu.SemaphoreType.DMA,
            pltpu.SemaphoreType.DMA,
            pltpu.SemaphoreType.DMA,
            pltpu.SemaphoreType.DMA,
        ],
    )


_BLK = 1000


def _root_body(x_ref, w_ref, b_ref, o_ref):
    o_ref[...] = (
        jnp.dot(x_ref[...], w_ref[...], preferred_element_type=jnp.float32)
        + b_ref[...]
    )


def _mid_body(acc_ref, r_ref, wrel_ref, o_ref):
    agg = acc_ref[0] + acc_ref[1]
    o_ref[...] = jnp.maximum(
        jnp.dot(agg, wrel_ref[...], preferred_element_type=jnp.float32)
        + r_ref[...],
        0.0,
    )


def _head_body(acc_ref, r_ref, wrel_ref, w3_ref, b3_ref, o_ref):
    agg = acc_ref[0] + acc_ref[1]
    h2 = jnp.maximum(
        jnp.dot(agg, wrel_ref[...], preferred_element_type=jnp.float32)
        + r_ref[...],
        0.0,
    )
    o_ref[...] = (
        jnp.dot(h2, w3_ref[...], preferred_element_type=jnp.float32)
        + b3_ref[...]
    )


def _row_spec(i):
    return (i, 0)


def _rep_spec(i):
    return (0, 0)


def _root(x, w, b):
    return pl.pallas_call(
        _root_body,
        grid=(_N // _BLK,),
        in_specs=[
            pl.BlockSpec((_BLK, _H), _row_spec),
            pl.BlockSpec((_H, _H), _rep_spec),
            pl.BlockSpec((1, _H), _rep_spec),
        ],
        out_specs=pl.BlockSpec((_BLK, _H), _row_spec),
        out_shape=jax.ShapeDtypeStruct((_N, _H), jnp.float32),
    )(x, w, b)


def _acc_spec(i):
    return (0, i, 0)


def _mid(acc, r, wrel):
    return pl.pallas_call(
        _mid_body,
        grid=(_N // _BLK,),
        in_specs=[
            pl.BlockSpec((_NC, _BLK, _H), _acc_spec),
            pl.BlockSpec((_BLK, _H), _row_spec),
            pl.BlockSpec((_H, _H), _rep_spec),
        ],
        out_specs=pl.BlockSpec((_BLK, _H), _row_spec),
        out_shape=jax.ShapeDtypeStruct((_N, _H), jnp.float32),
    )(acc, r, wrel)


def _head(acc, r, wrel, w3, b3):
    return pl.pallas_call(
        _head_body,
        grid=(_N // _BLK,),
        in_specs=[
            pl.BlockSpec((_NC, _BLK, _H), _acc_spec),
            pl.BlockSpec((_BLK, _H), _row_spec),
            pl.BlockSpec((_H, _H), _rep_spec),
            pl.BlockSpec((_H, _OUT), _rep_spec),
            pl.BlockSpec((1, _OUT), _rep_spec),
        ],
        out_specs=pl.BlockSpec((_BLK, _OUT), _row_spec),
        out_shape=jax.ShapeDtypeStruct((_N, _OUT), jnp.float32),
    )(acc, r, wrel, w3, b3)


def kernel(x, edge_index, W1_rel, b1, W1_root, W2_rel, b2, W2_root, W3, b3):
    npad = _EPAD - _E
    src = jnp.concatenate(
        [edge_index[0], jnp.zeros((npad,), jnp.int32)]
    ).reshape(_NW, _CHUNKS, _CHUNK)
    dst = jnp.concatenate(
        [edge_index[1], jnp.full((npad,), _DUMP, jnp.int32)]
    ).reshape(_NW, _CHUNKS, _CHUNK)
    zblk = jnp.zeros((_ZROWS, _D), jnp.float32)

    seg = _seg_sum()
    r1 = _root(x, W1_root, b1.reshape(1, _H))
    acc1 = seg(x, src, dst, zblk)
    h1 = _mid(acc1, r1, W1_rel)
    r2 = _root(h1, W2_root, b2.reshape(1, _H))
    acc2 = seg(h1, src, dst, zblk)
    return _head(acc2, r2, W2_rel, W3, b3.reshape(1, _OUT))


# speedup vs baseline: 1.0001x; 1.0001x over previous
import functools

import jax
import jax.numpy as jnp
from jax import lax
from jax.experimental import pallas as pl
from jax.experimental.pallas import tpu as pltpu
from jax.experimental.pallas import tpu_sc as plsc

_N = 10000
_D = 128
_H = 128
_OUT = 192
_E = 320000

_NC = 2
_NS = 16
_NW = _NC * _NS
_CHUNK = 128
_CHUNKS = 80
_EPAD = _NW * _CHUNKS * _CHUNK
_NDUMP = 128
_NACC = _N + _NDUMP
_HCH = 16
_STAGES = _CHUNKS // _HCH
_ZROWS = 400
_ZTASKS = _N // _ZROWS

def _seg_sum_body(x_hbm, src_hbm, dst_hbm, zblk_hbm, out_hbm,
                  acc, src_v, dst_v, rows_a, rows_b,
                  sem_a, sem_b, sem_sa, sem_sb):
    cid = lax.axis_index("c")
    sid = lax.axis_index("s")
    wid = cid * _NS + sid

    pltpu.sync_copy(zblk_hbm, acc.at[pl.ds(sid * _ZROWS, _ZROWS)])

    @pl.when(sid < _ZTASKS - _NS)
    def _():
        pltpu.sync_copy(zblk_hbm,
                        acc.at[pl.ds((sid + _NS) * _ZROWS, _ZROWS)])

    plsc.subcore_barrier()

    for h in range(_STAGES):
        pltpu.sync_copy(src_hbm.at[wid, pl.ds(h * _HCH, _HCH)], src_v)
        pltpu.sync_copy(dst_hbm.at[wid, pl.ds(h * _HCH, _HCH)], dst_v)
        pltpu.async_copy(x_hbm.at[src_v.at[0]], rows_a, sem_a)
        pltpu.async_copy(x_hbm.at[src_v.at[1]], rows_b, sem_b)

        def body(i, carry):
            j = i * 2
            pltpu.make_async_copy(x_hbm.at[src_v.at[j]], rows_a, sem_a).wait()
            pltpu.async_copy(rows_a, acc.at[dst_v.at[j]], sem_sa, add=True)

            @pl.when(j + 2 < _HCH)
            def _():
                pltpu.make_async_copy(rows_a, acc.at[dst_v.at[j]],
                                      sem_sa).wait()
                pltpu.async_copy(x_hbm.at[src_v.at[j + 2]], rows_a, sem_a)

            pltpu.make_async_copy(x_hbm.at[src_v.at[j + 1]], rows_b,
                                  sem_b).wait()
            pltpu.async_copy(rows_b, acc.at[dst_v.at[j + 1]], sem_sb,
                             add=True)

            @pl.when(j + 3 < _HCH)
            def _():
                pltpu.make_async_copy(rows_b, acc.at[dst_v.at[j + 1]],
                                      sem_sb).wait()
                pltpu.async_copy(x_hbm.at[src_v.at[j + 3]], rows_b, sem_b)

            return carry

        lax.fori_loop(0, _HCH // 2, body, 0)
        pltpu.make_async_copy(rows_a, acc.at[dst_v.at[0]], sem_sa).wait()
        pltpu.make_async_copy(rows_b, acc.at[dst_v.at[0]], sem_sb).wait()

    plsc.subcore_barrier()

    pltpu.sync_copy(acc.at[pl.ds(sid * _ZROWS, _ZROWS)],
                    out_hbm.at[cid, pl.ds(sid * _ZROWS, _ZROWS)])

    @pl.when(sid < _ZTASKS - _NS)
    def _():
        pltpu.sync_copy(acc.at[pl.ds((sid + _NS) * _ZROWS, _ZROWS)],
                        out_hbm.at[cid, pl.ds((sid + _NS) * _ZROWS, _ZROWS)])


@functools.cache
def _seg_sum():
    mesh = plsc.VectorSubcoreMesh(core_axis_name="c", subcore_axis_name="s",
                                  num_cores=_NC, num_subcores=_NS)
    return pl.kernel(
        _seg_sum_body,
        out_type=jax.ShapeDtypeStruct((_NC, _N, _D), jnp.float32),
        mesh=mesh,
        scratch_types=[
            pltpu.VMEM_SHARED((_NACC, _D), jnp.float32),
            pltpu.VMEM((_HCH, _CHUNK), jnp.int32),
            pltpu.VMEM((_HCH, _CHUNK), jnp.int32),
            pltpu.VMEM((_CHUNK, _D), jnp.float32),
            pltpu.VMEM((_CHUNK, _D), jnp.float32),
            pltpu.SemaphoreType.DMA,
            pltpu.SemaphoreType.DMA,
            pltpu.SemaphoreType.DMA,
            pltpu.SemaphoreType.DMA,
        ],
    )


_BLK = 1000


def _root_body(x_ref, w_ref, b_ref, o_ref):
    o_ref[...] = (
        jnp.dot(x_ref[...], w_ref[...], preferred_element_type=jnp.float32)
        + b_ref[...]
    )


def _mid_body(acc_ref, r_ref, wrel_ref, o_ref):
    agg = acc_ref[0] + acc_ref[1]
    o_ref[...] = jnp.maximum(
        jnp.dot(agg, wrel_ref[...], preferred_element_type=jnp.float32)
        + r_ref[...],
        0.0,
    )


def _head_body(acc_ref, r_ref, wrel_ref, w3_ref, b3_ref, o_ref):
    agg = acc_ref[0] + acc_ref[1]
    h2 = jnp.maximum(
        jnp.dot(agg, wrel_ref[...], preferred_element_type=jnp.float32)
        + r_ref[...],
        0.0,
    )
    o_ref[...] = (
        jnp.dot(h2, w3_ref[...], preferred_element_type=jnp.float32)
        + b3_ref[...]
    )


def _row_spec(i):
    return (i, 0)


def _rep_spec(i):
    return (0, 0)


def _root(x, w, b):
    return pl.pallas_call(
        _root_body,
        grid=(_N // _BLK,),
        in_specs=[
            pl.BlockSpec((_BLK, _H), _row_spec),
            pl.BlockSpec((_H, _H), _rep_spec),
            pl.BlockSpec((1, _H), _rep_spec),
        ],
        out_specs=pl.BlockSpec((_BLK, _H), _row_spec),
        out_shape=jax.ShapeDtypeStruct((_N, _H), jnp.float32),
    )(x, w, b)


def _acc_spec(i):
    return (0, i, 0)


def _mid(acc, r, wrel):
    return pl.pallas_call(
        _mid_body,
        grid=(_N // _BLK,),
        in_specs=[
            pl.BlockSpec((_NC, _BLK, _H), _acc_spec),
            pl.BlockSpec((_BLK, _H), _row_spec),
            pl.BlockSpec((_H, _H), _rep_spec),
        ],
        out_specs=pl.BlockSpec((_BLK, _H), _row_spec),
        out_shape=jax.ShapeDtypeStruct((_N, _H), jnp.float32),
    )(acc, r, wrel)


def _head(acc, r, wrel, w3, b3):
    return pl.pallas_call(
        _head_body,
        grid=(_N // _BLK,),
        in_specs=[
            pl.BlockSpec((_NC, _BLK, _H), _acc_spec),
            pl.BlockSpec((_BLK, _H), _row_spec),
            pl.BlockSpec((_H, _H), _rep_spec),
            pl.BlockSpec((_H, _OUT), _rep_spec),
            pl.BlockSpec((1, _OUT), _rep_spec),
        ],
        out_specs=pl.BlockSpec((_BLK, _OUT), _row_spec),
        out_shape=jax.ShapeDtypeStruct((_N, _OUT), jnp.float32),
    )(acc, r, wrel, w3, b3)


def kernel(x, edge_index, W1_rel, b1, W1_root, W2_rel, b2, W2_root, W3, b3):
    npad = _EPAD - _E
    src = jnp.concatenate(
        [edge_index[0], jnp.zeros((npad,), jnp.int32)]
    ).reshape(_NW, _CHUNKS, _CHUNK)
    pad_dst = _N + jnp.arange(npad, dtype=jnp.int32) % _NDUMP
    dst = jnp.concatenate(
        [edge_index[1], pad_dst]
    ).reshape(_NW, _CHUNKS, _CHUNK)
    zblk = jnp.zeros((_ZROWS, _D), jnp.float32)

    seg = _seg_sum()
    r1 = _root(x, W1_root, b1.reshape(1, _H))
    acc1 = seg(x, src, dst, zblk)
    h1 = _mid(acc1, r1, W1_rel)
    r2 = _root(h1, W2_root, b2.reshape(1, _H))
    acc2 = seg(h1, src, dst, zblk)
    return _head(acc2, r2, W2_rel, W3, b3.reshape(1, _OUT))


# speedup vs baseline: 3.2654x; 3.2650x over previous
import functools

import jax
import jax.numpy as jnp
from jax import lax
from jax.experimental import pallas as pl
from jax.experimental.pallas import tpu as pltpu
from jax.experimental.pallas import tpu_sc as plsc

_N = 10000
_D = 128
_H = 128
_OUT = 192
_E = 320000

_NC = 2
_NS = 16
_NW = _NC * _NS
_CHUNK = 128
_CHUNKS = 80
_EPAD = _NW * _CHUNKS * _CHUNK
_NDUMP = 128
_NACC = _N + _NDUMP
_HCH = 16
_STAGES = _CHUNKS // _HCH
_ZROWS = 400
_ZTASKS = _N // _ZROWS

def _seg_sum_body(x_hbm, src_hbm, dst_hbm, zblk_hbm, out_hbm,
                  acc, src_v, dst_v, rows_a, rows_b,
                  sem_a, sem_b, sem_sa, sem_sb):
    cid = lax.axis_index("c")
    sid = lax.axis_index("s")
    wid = cid * _NS + sid

    pltpu.sync_copy(zblk_hbm, acc.at[pl.ds(sid * _ZROWS, _ZROWS)])

    @pl.when(sid < _ZTASKS - _NS)
    def _():
        pltpu.sync_copy(zblk_hbm,
                        acc.at[pl.ds((sid + _NS) * _ZROWS, _ZROWS)])

    plsc.subcore_barrier()

    for h in range(_STAGES):
        pltpu.sync_copy(src_hbm.at[wid, pl.ds(h * _HCH, _HCH)], src_v)
        pltpu.sync_copy(dst_hbm.at[wid, pl.ds(h * _HCH, _HCH)], dst_v)
        pltpu.async_copy(x_hbm.at[src_v.at[0]], rows_a, sem_a)
        pltpu.async_copy(x_hbm.at[src_v.at[1]], rows_b, sem_b)

        def body(i, carry):
            j = i * 2
            pltpu.make_async_copy(x_hbm.at[src_v.at[j]], rows_a, sem_a).wait()
            pltpu.async_copy(rows_a, acc.at[dst_v.at[j]], sem_sa, add=True)

            @pl.when(j + 2 < _HCH)
            def _():
                pltpu.make_async_copy(rows_a, acc.at[dst_v.at[j]],
                                      sem_sa).wait()
                pltpu.async_copy(x_hbm.at[src_v.at[j + 2]], rows_a, sem_a)

            pltpu.make_async_copy(x_hbm.at[src_v.at[j + 1]], rows_b,
                                  sem_b).wait()
            pltpu.async_copy(rows_b, acc.at[dst_v.at[j + 1]], sem_sb,
                             add=True)

            @pl.when(j + 3 < _HCH)
            def _():
                pltpu.make_async_copy(rows_b, acc.at[dst_v.at[j + 1]],
                                      sem_sb).wait()
                pltpu.async_copy(x_hbm.at[src_v.at[j + 3]], rows_b, sem_b)

            return carry

        lax.fori_loop(0, _HCH // 2, body, 0)
        pltpu.make_async_copy(rows_a, acc.at[dst_v.at[0]], sem_sa).wait()
        pltpu.make_async_copy(rows_b, acc.at[dst_v.at[0]], sem_sb).wait()

    plsc.subcore_barrier()

    pltpu.sync_copy(acc.at[pl.ds(sid * _ZROWS, _ZROWS)],
                    out_hbm.at[cid, pl.ds(sid * _ZROWS, _ZROWS)])

    @pl.when(sid < _ZTASKS - _NS)
    def _():
        pltpu.sync_copy(acc.at[pl.ds((sid + _NS) * _ZROWS, _ZROWS)],
                        out_hbm.at[cid, pl.ds((sid + _NS) * _ZROWS, _ZROWS)])


@functools.cache
def _seg_sum():
    mesh = plsc.VectorSubcoreMesh(core_axis_name="c", subcore_axis_name="s",
                                  num_cores=_NC, num_subcores=_NS)
    return pl.kernel(
        _seg_sum_body,
        out_type=jax.ShapeDtypeStruct((_NC, _N, _D), jnp.float32),
        mesh=mesh,
        scratch_types=[
            pltpu.VMEM_SHARED((_NACC, _D), jnp.float32),
            pltpu.VMEM((_HCH, _CHUNK), jnp.int32),
            pltpu.VMEM((_HCH, _CHUNK), jnp.int32),
            pltpu.VMEM((_CHUNK, _D), jnp.float32),
            pltpu.VMEM((_CHUNK, _D), jnp.float32),
            pltpu.SemaphoreType.DMA,
            pltpu.SemaphoreType.DMA,
            pltpu.SemaphoreType.DMA,
            pltpu.SemaphoreType.DMA,
        ],
    )


_BLK = 1000


def _root_body(x_ref, w_ref, b_ref, o_ref):
    o_ref[...] = (
        jnp.dot(x_ref[...], w_ref[...], preferred_element_type=jnp.float32)
        + b_ref[...]
    )


def _mid_body(acc_ref, r_ref, wrel_ref, o_ref):
    agg = acc_ref[0] + acc_ref[1]
    o_ref[...] = jnp.maximum(
        jnp.dot(agg, wrel_ref[...], preferred_element_type=jnp.float32)
        + r_ref[...],
        0.0,
    )


def _head_body(acc_ref, r_ref, wrel_ref, w3_ref, b3_ref, o_ref):
    agg = acc_ref[0] + acc_ref[1]
    h2 = jnp.maximum(
        jnp.dot(agg, wrel_ref[...], preferred_element_type=jnp.float32)
        + r_ref[...],
        0.0,
    )
    o_ref[...] = (
        jnp.dot(h2, w3_ref[...], preferred_element_type=jnp.float32)
        + b3_ref[...]
    )


def _row_spec(i):
    return (i, 0)


def _rep_spec(i):
    return (0, 0)


def _root(x, w, b):
    return pl.pallas_call(
        _root_body,
        grid=(_N // _BLK,),
        in_specs=[
            pl.BlockSpec((_BLK, _H), _row_spec),
            pl.BlockSpec((_H, _H), _rep_spec),
            pl.BlockSpec((1, _H), _rep_spec),
        ],
        out_specs=pl.BlockSpec((_BLK, _H), _row_spec),
        out_shape=jax.ShapeDtypeStruct((_N, _H), jnp.float32),
    )(x, w, b)


def _acc_spec(i):
    return (0, i, 0)


def _mid(acc, r, wrel):
    return pl.pallas_call(
        _mid_body,
        grid=(_N // _BLK,),
        in_specs=[
            pl.BlockSpec((_NC, _BLK, _H), _acc_spec),
            pl.BlockSpec((_BLK, _H), _row_spec),
            pl.BlockSpec((_H, _H), _rep_spec),
        ],
        out_specs=pl.BlockSpec((_BLK, _H), _row_spec),
        out_shape=jax.ShapeDtypeStruct((_N, _H), jnp.float32),
    )(acc, r, wrel)


def _head(acc, r, wrel, w3, b3):
    return pl.pallas_call(
        _head_body,
        grid=(_N // _BLK,),
        in_specs=[
            pl.BlockSpec((_NC, _BLK, _H), _acc_spec),
            pl.BlockSpec((_BLK, _H), _row_spec),
            pl.BlockSpec((_H, _H), _rep_spec),
            pl.BlockSpec((_H, _OUT), _rep_spec),
            pl.BlockSpec((1, _OUT), _rep_spec),
        ],
        out_specs=pl.BlockSpec((_BLK, _OUT), _row_spec),
        out_shape=jax.ShapeDtypeStruct((_N, _OUT), jnp.float32),
    )(acc, r, wrel, w3, b3)


def kernel(x, edge_index, W1_rel, b1, W1_root, W2_rel, b2, W2_root, W3, b3):
    npad = _EPAD - _E
    pad_src = jnp.arange(npad, dtype=jnp.int32) % _N
    src = jnp.concatenate(
        [edge_index[0], pad_src]
    ).reshape(_NW, _CHUNKS, _CHUNK)
    pad_dst = _N + jnp.arange(npad, dtype=jnp.int32) % _NDUMP
    dst = jnp.concatenate(
        [edge_index[1], pad_dst]
    ).reshape(_NW, _CHUNKS, _CHUNK)
    zblk = jnp.zeros((_ZROWS, _D), jnp.float32)

    seg = _seg_sum()
    r1 = _root(x, W1_root, b1.reshape(1, _H))
    acc1 = seg(x, src, dst, zblk)
    h1 = _mid(acc1, r1, W1_rel)
    r2 = _root(h1, W2_root, b2.reshape(1, _H))
    acc2 = seg(h1, src, dst, zblk)
    return _head(acc2, r2, W2_rel, W3, b3.reshape(1, _OUT))


# speedup vs baseline: 3.4503x; 1.0566x over previous
import functools

import jax
import jax.numpy as jnp
from jax import lax
from jax.experimental import pallas as pl
from jax.experimental.pallas import tpu as pltpu
from jax.experimental.pallas import tpu_sc as plsc

_N = 10000
_D = 128
_H = 128
_OUT = 192
_E = 320000

_NC = 2
_NS = 16
_NW = _NC * _NS
_EPT = _E // _NW
_CHUNK = 125
_CHUNKS = _EPT // _CHUNK
_HCH = _CHUNKS // 2
_STAGES = 2
_ZROWS = 400
_ZTASKS = _N // _ZROWS

def _seg_sum_body(x_hbm, src_hbm, dst_hbm, zblk_hbm, out_hbm,
                  acc, src_v, dst_v, rows_a, rows_b,
                  sem_a, sem_b, sem_sa, sem_sb):
    cid = lax.axis_index("c")
    sid = lax.axis_index("s")
    wid = cid * _NS + sid

    pltpu.sync_copy(zblk_hbm, acc.at[pl.ds(sid * _ZROWS, _ZROWS)])

    @pl.when(sid < _ZTASKS - _NS)
    def _():
        pltpu.sync_copy(zblk_hbm,
                        acc.at[pl.ds((sid + _NS) * _ZROWS, _ZROWS)])

    plsc.subcore_barrier()

    for h in range(_STAGES):
        pltpu.sync_copy(src_hbm.at[wid, pl.ds(h * _HCH, _HCH)], src_v)
        pltpu.sync_copy(dst_hbm.at[wid, pl.ds(h * _HCH, _HCH)], dst_v)
        pltpu.async_copy(x_hbm.at[src_v.at[0]], rows_a, sem_a)
        pltpu.async_copy(x_hbm.at[src_v.at[1]], rows_b, sem_b)

        def body(i, carry):
            j = i * 2
            pltpu.make_async_copy(x_hbm.at[src_v.at[j]], rows_a, sem_a).wait()
            pltpu.async_copy(rows_a, acc.at[dst_v.at[j]], sem_sa, add=True)

            @pl.when(j + 2 < _HCH)
            def _():
                pltpu.make_async_copy(rows_a, acc.at[dst_v.at[j]],
                                      sem_sa).wait()
                pltpu.async_copy(x_hbm.at[src_v.at[j + 2]], rows_a, sem_a)

            pltpu.make_async_copy(x_hbm.at[src_v.at[j + 1]], rows_b,
                                  sem_b).wait()
            pltpu.async_copy(rows_b, acc.at[dst_v.at[j + 1]], sem_sb,
                             add=True)

            @pl.when(j + 3 < _HCH)
            def _():
                pltpu.make_async_copy(rows_b, acc.at[dst_v.at[j + 1]],
                                      sem_sb).wait()
                pltpu.async_copy(x_hbm.at[src_v.at[j + 3]], rows_b, sem_b)

            return carry

        lax.fori_loop(0, _HCH // 2, body, 0)
        pltpu.make_async_copy(rows_a, acc.at[dst_v.at[0]], sem_sa).wait()
        pltpu.make_async_copy(rows_b, acc.at[dst_v.at[0]], sem_sb).wait()

    plsc.subcore_barrier()

    pltpu.sync_copy(acc.at[pl.ds(sid * _ZROWS, _ZROWS)],
                    out_hbm.at[cid, pl.ds(sid * _ZROWS, _ZROWS)])

    @pl.when(sid < _ZTASKS - _NS)
    def _():
        pltpu.sync_copy(acc.at[pl.ds((sid + _NS) * _ZROWS, _ZROWS)],
                        out_hbm.at[cid, pl.ds((sid + _NS) * _ZROWS, _ZROWS)])


@functools.cache
def _seg_sum():
    mesh = plsc.VectorSubcoreMesh(core_axis_name="c", subcore_axis_name="s",
                                  num_cores=_NC, num_subcores=_NS)
    return pl.kernel(
        _seg_sum_body,
        out_type=jax.ShapeDtypeStruct((_NC, _N, _D), jnp.float32),
        mesh=mesh,
        scratch_types=[
            pltpu.VMEM_SHARED((_N, _D), jnp.float32),
            pltpu.VMEM((_HCH, _CHUNK), jnp.int32),
            pltpu.VMEM((_HCH, _CHUNK), jnp.int32),
            pltpu.VMEM((_CHUNK, _D), jnp.float32),
            pltpu.VMEM((_CHUNK, _D), jnp.float32),
            pltpu.SemaphoreType.DMA,
            pltpu.SemaphoreType.DMA,
            pltpu.SemaphoreType.DMA,
            pltpu.SemaphoreType.DMA,
        ],
    )


_BLK = 1000


def _root_body(x_ref, w_ref, b_ref, o_ref):
    o_ref[...] = (
        jnp.dot(x_ref[...], w_ref[...], preferred_element_type=jnp.float32)
        + b_ref[...]
    )


def _mid_body(acc_ref, r_ref, wrel_ref, o_ref):
    agg = acc_ref[0] + acc_ref[1]
    o_ref[...] = jnp.maximum(
        jnp.dot(agg, wrel_ref[...], preferred_element_type=jnp.float32)
        + r_ref[...],
        0.0,
    )


def _head_body(acc_ref, r_ref, wrel_ref, w3_ref, b3_ref, o_ref):
    agg = acc_ref[0] + acc_ref[1]
    h2 = jnp.maximum(
        jnp.dot(agg, wrel_ref[...], preferred_element_type=jnp.float32)
        + r_ref[...],
        0.0,
    )
    o_ref[...] = (
        jnp.dot(h2, w3_ref[...], preferred_element_type=jnp.float32)
        + b3_ref[...]
    )


def _row_spec(i):
    return (i, 0)


def _rep_spec(i):
    return (0, 0)


def _root(x, w, b):
    return pl.pallas_call(
        _root_body,
        grid=(_N // _BLK,),
        in_specs=[
            pl.BlockSpec((_BLK, _H), _row_spec),
            pl.BlockSpec((_H, _H), _rep_spec),
            pl.BlockSpec((1, _H), _rep_spec),
        ],
        out_specs=pl.BlockSpec((_BLK, _H), _row_spec),
        out_shape=jax.ShapeDtypeStruct((_N, _H), jnp.float32),
    )(x, w, b)


def _acc_spec(i):
    return (0, i, 0)


def _mid(acc, r, wrel):
    return pl.pallas_call(
        _mid_body,
        grid=(_N // _BLK,),
        in_specs=[
            pl.BlockSpec((_NC, _BLK, _H), _acc_spec),
            pl.BlockSpec((_BLK, _H), _row_spec),
            pl.BlockSpec((_H, _H), _rep_spec),
        ],
        out_specs=pl.BlockSpec((_BLK, _H), _row_spec),
        out_shape=jax.ShapeDtypeStruct((_N, _H), jnp.float32),
    )(acc, r, wrel)


def _head(acc, r, wrel, w3, b3):
    return pl.pallas_call(
        _head_body,
        grid=(_N // _BLK,),
        in_specs=[
            pl.BlockSpec((_NC, _BLK, _H), _acc_spec),
            pl.BlockSpec((_BLK, _H), _row_spec),
            pl.BlockSpec((_H, _H), _rep_spec),
            pl.BlockSpec((_H, _OUT), _rep_spec),
            pl.BlockSpec((1, _OUT), _rep_spec),
        ],
        out_specs=pl.BlockSpec((_BLK, _OUT), _row_spec),
        out_shape=jax.ShapeDtypeStruct((_N, _OUT), jnp.float32),
    )(acc, r, wrel, w3, b3)


def kernel(x, edge_index, W1_rel, b1, W1_root, W2_rel, b2, W2_root, W3, b3):
    src = edge_index[0].reshape(_NW, _CHUNKS, _CHUNK)
    dst = edge_index[1].reshape(_NW, _CHUNKS, _CHUNK)
    zblk = jnp.zeros((_ZROWS, _D), jnp.float32)

    seg = _seg_sum()
    r1 = _root(x, W1_root, b1.reshape(1, _H))
    acc1 = seg(x, src, dst, zblk)
    h1 = _mid(acc1, r1, W1_rel)
    r2 = _root(h1, W2_root, b2.reshape(1, _H))
    acc2 = seg(h1, src, dst, zblk)
    return _head(acc2, r2, W2_rel, W3, b3.reshape(1, _OUT))


# speedup vs baseline: 3.5775x; 1.0369x over previous
import functools

import jax
import jax.numpy as jnp
from jax import lax
from jax.experimental import pallas as pl
from jax.experimental.pallas import tpu as pltpu
from jax.experimental.pallas import tpu_sc as plsc

_N = 10000
_D = 128
_H = 128
_OUT = 192
_E = 320000

_NC = 2
_NS = 16
_NW = _NC * _NS
_EPT = _E // _NW
_CHUNK = 125
_CHUNKS = _EPT // _CHUNK
_HCH = _CHUNKS // 2
_STAGES = 2
_ZROWS = 400
_ZTASKS = _N // _ZROWS

def _seg_sum_body(x_hbm, src_hbm, dst_hbm, out_hbm,
                  acc, src_v, dst_v, rows_a, rows_b,
                  sem_a, sem_b, sem_sa, sem_sb):
    cid = lax.axis_index("c")
    sid = lax.axis_index("s")
    wid = cid * _NS + sid

    zv = jnp.zeros((16,), jnp.float32)

    def zfill(i, carry):
        rows_a[i // 8, pl.ds((i % 8) * 16, 16)] = zv
        return carry

    lax.fori_loop(0, _CHUNK * 8, zfill, 0)
    for z in range(5):
        pltpu.sync_copy(rows_a,
                        acc.at[pl.ds(sid * 625 + z * _CHUNK, _CHUNK)])

    plsc.subcore_barrier()

    for h in range(_STAGES):
        pltpu.sync_copy(src_hbm.at[wid, pl.ds(h * _HCH, _HCH)], src_v)
        pltpu.sync_copy(dst_hbm.at[wid, pl.ds(h * _HCH, _HCH)], dst_v)
        pltpu.async_copy(x_hbm.at[src_v.at[0]], rows_a, sem_a)
        pltpu.async_copy(x_hbm.at[src_v.at[1]], rows_b, sem_b)

        def body(i, carry):
            j = i * 2
            pltpu.make_async_copy(x_hbm.at[src_v.at[j]], rows_a, sem_a).wait()
            pltpu.async_copy(rows_a, acc.at[dst_v.at[j]], sem_sa, add=True)

            @pl.when(j + 2 < _HCH)
            def _():
                pltpu.make_async_copy(rows_a, acc.at[dst_v.at[j]],
                                      sem_sa).wait()
                pltpu.async_copy(x_hbm.at[src_v.at[j + 2]], rows_a, sem_a)

            pltpu.make_async_copy(x_hbm.at[src_v.at[j + 1]], rows_b,
                                  sem_b).wait()
            pltpu.async_copy(rows_b, acc.at[dst_v.at[j + 1]], sem_sb,
                             add=True)

            @pl.when(j + 3 < _HCH)
            def _():
                pltpu.make_async_copy(rows_b, acc.at[dst_v.at[j + 1]],
                                      sem_sb).wait()
                pltpu.async_copy(x_hbm.at[src_v.at[j + 3]], rows_b, sem_b)

            return carry

        lax.fori_loop(0, _HCH // 2, body, 0)
        pltpu.make_async_copy(rows_a, acc.at[dst_v.at[0]], sem_sa).wait()
        pltpu.make_async_copy(rows_b, acc.at[dst_v.at[0]], sem_sb).wait()

    plsc.subcore_barrier()

    pltpu.async_copy(acc.at[pl.ds(sid * _ZROWS, _ZROWS)],
                     out_hbm.at[cid, pl.ds(sid * _ZROWS, _ZROWS)], sem_a)

    @pl.when(sid < _ZTASKS - _NS)
    def _():
        pltpu.async_copy(acc.at[pl.ds((sid + _NS) * _ZROWS, _ZROWS)],
                         out_hbm.at[cid, pl.ds((sid + _NS) * _ZROWS, _ZROWS)],
                         sem_b)

    pltpu.make_async_copy(acc.at[pl.ds(sid * _ZROWS, _ZROWS)],
                          out_hbm.at[cid, pl.ds(sid * _ZROWS, _ZROWS)],
                          sem_a).wait()

    @pl.when(sid < _ZTASKS - _NS)
    def _():
        pltpu.make_async_copy(
            acc.at[pl.ds((sid + _NS) * _ZROWS, _ZROWS)],
            out_hbm.at[cid, pl.ds((sid + _NS) * _ZROWS, _ZROWS)],
            sem_b).wait()


@functools.cache
def _seg_sum():
    mesh = plsc.VectorSubcoreMesh(core_axis_name="c", subcore_axis_name="s",
                                  num_cores=_NC, num_subcores=_NS)
    return pl.kernel(
        _seg_sum_body,
        out_type=jax.ShapeDtypeStruct((_NC, _N, _D), jnp.float32),
        mesh=mesh,
        scratch_types=[
            pltpu.VMEM_SHARED((_N, _D), jnp.float32),
            pltpu.VMEM((_HCH, _CHUNK), jnp.int32),
            pltpu.VMEM((_HCH, _CHUNK), jnp.int32),
            pltpu.VMEM((_CHUNK, _D), jnp.float32),
            pltpu.VMEM((_CHUNK, _D), jnp.float32),
            pltpu.SemaphoreType.DMA,
            pltpu.SemaphoreType.DMA,
            pltpu.SemaphoreType.DMA,
            pltpu.SemaphoreType.DMA,
        ],
    )


_BLK = 1000


def _root_body(x_ref, w_ref, b_ref, o_ref):
    o_ref[...] = (
        jnp.dot(x_ref[...], w_ref[...], preferred_element_type=jnp.float32)
        + b_ref[...]
    )


def _mid_body(acc_ref, r_ref, wrel_ref, o_ref):
    agg = acc_ref[0] + acc_ref[1]
    o_ref[...] = jnp.maximum(
        jnp.dot(agg, wrel_ref[...], preferred_element_type=jnp.float32)
        + r_ref[...],
        0.0,
    )


def _head_body(acc_ref, r_ref, wrel_ref, w3_ref, b3_ref, o_ref):
    agg = acc_ref[0] + acc_ref[1]
    h2 = jnp.maximum(
        jnp.dot(agg, wrel_ref[...], preferred_element_type=jnp.float32)
        + r_ref[...],
        0.0,
    )
    o_ref[...] = (
        jnp.dot(h2, w3_ref[...], preferred_element_type=jnp.float32)
        + b3_ref[...]
    )


def _row_spec(i):
    return (i, 0)


def _rep_spec(i):
    return (0, 0)


def _root(x, w, b):
    return pl.pallas_call(
        _root_body,
        grid=(_N // _BLK,),
        in_specs=[
            pl.BlockSpec((_BLK, _H), _row_spec),
            pl.BlockSpec((_H, _H), _rep_spec),
            pl.BlockSpec((1, _H), _rep_spec),
        ],
        out_specs=pl.BlockSpec((_BLK, _H), _row_spec),
        out_shape=jax.ShapeDtypeStruct((_N, _H), jnp.float32),
    )(x, w, b)


def _acc_spec(i):
    return (0, i, 0)


def _mid(acc, r, wrel):
    return pl.pallas_call(
        _mid_body,
        grid=(_N // _BLK,),
        in_specs=[
            pl.BlockSpec((_NC, _BLK, _H), _acc_spec),
            pl.BlockSpec((_BLK, _H), _row_spec),
            pl.BlockSpec((_H, _H), _rep_spec),
        ],
        out_specs=pl.BlockSpec((_BLK, _H), _row_spec),
        out_shape=jax.ShapeDtypeStruct((_N, _H), jnp.float32),
    )(acc, r, wrel)


def _head(acc, r, wrel, w3, b3):
    return pl.pallas_call(
        _head_body,
        grid=(_N // _BLK,),
        in_specs=[
            pl.BlockSpec((_NC, _BLK, _H), _acc_spec),
            pl.BlockSpec((_BLK, _H), _row_spec),
            pl.BlockSpec((_H, _H), _rep_spec),
            pl.BlockSpec((_H, _OUT), _rep_spec),
            pl.BlockSpec((1, _OUT), _rep_spec),
        ],
        out_specs=pl.BlockSpec((_BLK, _OUT), _row_spec),
        out_shape=jax.ShapeDtypeStruct((_N, _OUT), jnp.float32),
    )(acc, r, wrel, w3, b3)


def kernel(x, edge_index, W1_rel, b1, W1_root, W2_rel, b2, W2_root, W3, b3):
    src = edge_index[0].reshape(_NW, _CHUNKS, _CHUNK)
    dst = edge_index[1].reshape(_NW, _CHUNKS, _CHUNK)
    seg = _seg_sum()
    r1 = _root(x, W1_root, b1.reshape(1, _H))
    acc1 = seg(x, src, dst)
    h1 = _mid(acc1, r1, W1_rel)
    r2 = _root(h1, W2_root, b2.reshape(1, _H))
    acc2 = seg(h1, src, dst)
    return _head(acc2, r2, W2_rel, W3, b3.reshape(1, _OUT))
